# initial kernel scaffold (unmeasured)
import jax
import jax.numpy as jnp
from jax import lax
from jax.experimental import pallas as pl
from jax.experimental.pallas import tpu as pltpu

N_DEV = 4
B = 2
S = 4096
C = 1024
O = 1024
TAPS = 4
CHUNK = 512
SLAB = S // N_DEV


def kernel(x, k, Wp):
    def body(x_hbm, k_ref, w_ref, out_ref,
             xs_ref, comm_ref, x_sem, send_sem, recv_sem, credit_sem):
        me = lax.axis_index("i")
        left = lax.rem(me + N_DEV - 1, N_DEV)
        right = lax.rem(me + 1, N_DEV)

        barrier = pltpu.get_barrier_semaphore()
        for nbr in (left, right):
            pl.semaphore_signal(barrier, inc=1, device_id=(nbr,),
                                device_id_type=pl.DeviceIdType.MESH)
        pl.semaphore_wait(barrier, 2)

        for b in range(B):
            for c0 in range(0, S, CHUNK):
                if c0 == 0:
                    cp = pltpu.make_async_copy(
                        x_hbm.at[b, pl.ds(0, CHUNK), :],
                        xs_ref.at[pl.ds(TAPS - 1, CHUNK), :],
                        x_sem,
                    )
                    cp.start()
                    xs_ref[0:TAPS - 1, :] = jnp.zeros((TAPS - 1, C), jnp.float32)
                    cp.wait()
                else:
                    cp = pltpu.make_async_copy(
                        x_hbm.at[b, pl.ds(c0 - (TAPS - 1), CHUNK + TAPS - 1), :],
                        xs_ref,
                        x_sem,
                    )
                    cp.start()
                    cp.wait()
                conv = xs_ref[TAPS - 1:TAPS - 1 + CHUNK, :] * k_ref[TAPS - 1:TAPS, :]
                for t in range(TAPS - 1):
                    conv += xs_ref[t:t + CHUNK, :] * k_ref[t:t + 1, :]
                a = conv * (1.0 / (1.0 + jnp.exp(-conv)))
                out_ref[b, c0:c0 + CHUNK, :] = jnp.dot(
                    a, w_ref[...], preferred_element_type=jnp.float32)

        def hop(step, send_idx, recv_idx, accumulate):
            if step > 0:
                pl.semaphore_wait(credit_sem, 1)
            rdma = pltpu.make_async_remote_copy(
                src_ref=out_ref.at[:, pl.ds(send_idx * SLAB, SLAB), :],
                dst_ref=comm_ref,
                send_sem=send_sem,
                recv_sem=recv_sem,
                device_id=(right,),
                device_id_type=pl.DeviceIdType.MESH,
            )
            rdma.start()
            rdma.wait()
            if accumulate:
                out_ref[:, pl.ds(recv_idx * SLAB, SLAB), :] += comm_ref[...]
            else:
                out_ref[:, pl.ds(recv_idx * SLAB, SLAB), :] = comm_ref[...]
            if step < 2 * (N_DEV - 1) - 1:
                pl.semaphore_signal(credit_sem, inc=1, device_id=(left,),
                                    device_id_type=pl.DeviceIdType.MESH)

        for s in range(N_DEV - 1):
            hop(s,
                lax.rem(me + (N_DEV - s), N_DEV),
                lax.rem(me + (N_DEV - s - 1), N_DEV),
                accumulate=True)
        for g in range(N_DEV - 1):
            hop(N_DEV - 1 + g,
                lax.rem(me + 1 + (N_DEV - g), N_DEV),
                lax.rem(me + (N_DEV - g), N_DEV),
                accumulate=False)

    return pl.pallas_call(
        body,
        out_shape=jax.ShapeDtypeStruct((B, S, O), jnp.float32),
        in_specs=[
            pl.BlockSpec(memory_space=pltpu.MemorySpace.HBM),
            pl.BlockSpec(memory_space=pltpu.MemorySpace.VMEM),
            pl.BlockSpec(memory_space=pltpu.MemorySpace.VMEM),
        ],
        out_specs=pl.BlockSpec(memory_space=pltpu.MemorySpace.VMEM),
        scratch_shapes=[
            pltpu.VMEM((CHUNK + TAPS - 1, C), jnp.float32),
            pltpu.VMEM((B, SLAB, O), jnp.float32),
            pltpu.SemaphoreType.DMA,
            pltpu.SemaphoreType.DMA,
            pltpu.SemaphoreType.DMA,
            pltpu.SemaphoreType.REGULAR,
        ],
        compiler_params=pltpu.CompilerParams(collective_id=0),
    )(x, k, Wp)


# baseline (device time: 698259 ns/iter reference)
import jax
import jax.numpy as jnp
from jax import lax
from jax.experimental import pallas as pl
from jax.experimental.pallas import tpu as pltpu

N_DEV = 4
B = 2
S = 4096
C = 1024
O = 1024
TAPS = 4
HALO = 8
CHUNK = 512
SLAB = S // N_DEV
N_STEPS = 2 * (N_DEV - 1)


def kernel(x, k, Wp):
    def body(x_hbm, k_ref, w_ref, out_hbm,
             xs_ref, st_ref, acc_ref, comm_ref,
             x_sem, st_sem, acc_sem, send_sem, recv_sem, credit_sem):
        me = lax.axis_index("i")
        left = lax.rem(me + N_DEV - 1, N_DEV)
        right = lax.rem(me + 1, N_DEV)

        barrier = pltpu.get_barrier_semaphore()
        for nbr in (left, right):
            pl.semaphore_signal(barrier, inc=1, device_id=(nbr,),
                                device_id_type=pl.DeviceIdType.MESH)
        pl.semaphore_wait(barrier, 2)

        for b in range(B):
            for c0 in range(0, S, CHUNK):
                if c0 == 0:
                    cp = pltpu.make_async_copy(
                        x_hbm.at[b, pl.ds(0, CHUNK), :],
                        xs_ref.at[pl.ds(HALO, CHUNK), :],
                        x_sem,
                    )
                    cp.start()
                    xs_ref[0:HALO, :] = jnp.zeros((HALO, C), jnp.float32)
                    cp.wait()
                else:
                    cp = pltpu.make_async_copy(
                        x_hbm.at[b, pl.ds(c0 - HALO, CHUNK + HALO), :],
                        xs_ref,
                        x_sem,
                    )
                    cp.start()
                    cp.wait()
                off = HALO - (TAPS - 1)
                conv = xs_ref[off + TAPS - 1:off + TAPS - 1 + CHUNK, :] \
                    * k_ref[TAPS - 1:TAPS, :]
                for t in range(TAPS - 1):
                    conv += xs_ref[off + t:off + t + CHUNK, :] * k_ref[t:t + 1, :]
                a = conv * (1.0 / (1.0 + jnp.exp(-conv)))
                st_ref[...] = jnp.dot(
                    a, w_ref[...], preferred_element_type=jnp.float32)
                cp = pltpu.make_async_copy(
                    st_ref, out_hbm.at[b, pl.ds(c0, CHUNK), :], st_sem)
                cp.start()
                cp.wait()

        def hop(step, send_idx, recv_idx, accumulate):
            if step > 0:
                pl.semaphore_wait(credit_sem, 1)
            rdma = pltpu.make_async_remote_copy(
                src_ref=out_hbm.at[:, pl.ds(send_idx * SLAB, SLAB), :],
                dst_ref=comm_ref,
                send_sem=send_sem,
                recv_sem=recv_sem,
                device_id=(right,),
                device_id_type=pl.DeviceIdType.MESH,
            )
            rdma.start()
            rdma.wait()
            dst_slab = out_hbm.at[:, pl.ds(recv_idx * SLAB, SLAB), :]
            if accumulate:
                cp = pltpu.make_async_copy(dst_slab, acc_ref, acc_sem)
                cp.start()
                cp.wait()
                acc_ref[...] += comm_ref[...]
                cp = pltpu.make_async_copy(acc_ref, dst_slab, acc_sem)
            else:
                cp = pltpu.make_async_copy(comm_ref, dst_slab, acc_sem)
            cp.start()
            cp.wait()
            if step < N_STEPS - 1:
                pl.semaphore_signal(credit_sem, inc=1, device_id=(left,),
                                    device_id_type=pl.DeviceIdType.MESH)

        for s in range(N_DEV - 1):
            hop(s,
                lax.rem(me + (N_DEV - s), N_DEV),
                lax.rem(me + (N_DEV - s - 1), N_DEV),
                accumulate=True)
        for g in range(N_DEV - 1):
            hop(N_DEV - 1 + g,
                lax.rem(me + 1 + (N_DEV - g), N_DEV),
                lax.rem(me + (N_DEV - g), N_DEV),
                accumulate=False)

    return pl.pallas_call(
        body,
        out_shape=jax.ShapeDtypeStruct((B, S, O), jnp.float32),
        in_specs=[
            pl.BlockSpec(memory_space=pltpu.MemorySpace.HBM),
            pl.BlockSpec(memory_space=pltpu.MemorySpace.VMEM),
            pl.BlockSpec(memory_space=pltpu.MemorySpace.VMEM),
        ],
        out_specs=pl.BlockSpec(memory_space=pltpu.MemorySpace.HBM),
        scratch_shapes=[
            pltpu.VMEM((CHUNK + HALO, C), jnp.float32),
            pltpu.VMEM((CHUNK, O), jnp.float32),
            pltpu.VMEM((B, SLAB, O), jnp.float32),
            pltpu.VMEM((B, SLAB, O), jnp.float32),
            pltpu.SemaphoreType.DMA,
            pltpu.SemaphoreType.DMA,
            pltpu.SemaphoreType.DMA,
            pltpu.SemaphoreType.DMA,
            pltpu.SemaphoreType.DMA,
            pltpu.SemaphoreType.REGULAR,
        ],
        compiler_params=pltpu.CompilerParams(
            collective_id=0,
            vmem_limit_bytes=56 * 1024 * 1024,
        ),
    )(x, k, Wp)


# device time: 327899 ns/iter; 2.1295x vs baseline; 2.1295x over previous
import jax
import jax.numpy as jnp
from jax import lax
from jax.experimental import pallas as pl
from jax.experimental.pallas import tpu as pltpu

N_DEV = 4
B = 2
S = 4096
C = 1024
O = 1024
TAPS = 4
HALO = 8
CHUNK = 512
SLAB = S // N_DEV


def kernel(x, k, Wp):
    def body(x_hbm, k_ref, w_ref, out_hbm,
             xs_ref, slab_r, slab_l, comm_r, comm_l,
             x_sem, out_r_sem, out_l_sem,
             send_r_sem, recv_r_sem, send_l_sem, recv_l_sem,
             credit_r, credit_l):
        me = lax.axis_index("i")
        left = lax.rem(me + N_DEV - 1, N_DEV)
        right = lax.rem(me + 1, N_DEV)

        def compute_slab(b, j, dst_ref):
            @pl.when(j == 0)
            def _():
                cp = pltpu.make_async_copy(
                    x_hbm.at[b, pl.ds(0, SLAB), :],
                    xs_ref.at[pl.ds(HALO, SLAB), :],
                    x_sem,
                )
                cp.start()
                xs_ref[0:HALO, :] = jnp.zeros((HALO, C), jnp.float32)
                cp.wait()

            @pl.when(j != 0)
            def _():
                cp = pltpu.make_async_copy(
                    x_hbm.at[b, pl.ds(j * SLAB - HALO, SLAB + HALO), :],
                    xs_ref,
                    x_sem,
                )
                cp.start()
                cp.wait()

            off = HALO - (TAPS - 1)
            for q in range(0, SLAB, CHUNK):
                conv = xs_ref[off + q + TAPS - 1:off + q + TAPS - 1 + CHUNK, :] \
                    * k_ref[TAPS - 1:TAPS, :]
                for t in range(TAPS - 1):
                    conv += xs_ref[off + q + t:off + q + t + CHUNK, :] \
                        * k_ref[t:t + 1, :]
                a = conv * (1.0 / (1.0 + jnp.exp(-conv)))
                dst_ref[q:q + CHUNK, :] = jnp.dot(
                    a, w_ref[...], preferred_element_type=jnp.float32)

        def ring_send(src_ref, dst_ref, send_sem, recv_sem, to):
            return pltpu.make_async_remote_copy(
                src_ref=src_ref, dst_ref=dst_ref,
                send_sem=send_sem, recv_sem=recv_sem,
                device_id=(to,), device_id_type=pl.DeviceIdType.MESH,
            )

        def send_credits():
            pl.semaphore_signal(credit_r, inc=1, device_id=(left,),
                                device_id_type=pl.DeviceIdType.MESH)
            pl.semaphore_signal(credit_l, inc=1, device_id=(right,),
                                device_id_type=pl.DeviceIdType.MESH)

        def wait_credits():
            pl.semaphore_wait(credit_r, 1)
            pl.semaphore_wait(credit_l, 1)

        barrier = pltpu.get_barrier_semaphore()
        for nbr in (left, right):
            pl.semaphore_signal(barrier, inc=1, device_id=(nbr,),
                                device_id_type=pl.DeviceIdType.MESH)
        pl.semaphore_wait(barrier, 2)

        compute_slab(0, me, slab_r.at[0])
        compute_slab(1, me, slab_l.at[0])

        for s in range(N_DEV - 1):
            if s >= 2:
                wait_credits()
            rr = ring_send(slab_r.at[s % 2], comm_r.at[s % 2],
                           send_r_sem, recv_r_sem, right)
            rl = ring_send(slab_l.at[s % 2], comm_l.at[s % 2],
                           send_l_sem, recv_l_sem, left)
            rr.start()
            rl.start()
            compute_slab(0, lax.rem(me + (N_DEV - s - 1), N_DEV),
                         slab_r.at[(s + 1) % 2])
            compute_slab(1, lax.rem(me + s + 1, N_DEV),
                         slab_l.at[(s + 1) % 2])
            rr.wait()
            rl.wait()
            nxt = (s + 1) % 2
            slab_r[nxt, :, :] = slab_r[nxt, :, :] + comm_r[s % 2, :, :]
            slab_l[nxt, :, :] = slab_l[nxt, :, :] + comm_l[s % 2, :, :]
            send_credits()

        for g in range(N_DEV - 1):
            s = N_DEV - 1 + g
            wait_credits()
            if g == 0:
                src_r, src_l = slab_r.at[1], slab_l.at[1]
            else:
                src_r, src_l = comm_r.at[(s - 1) % 2], comm_l.at[(s - 1) % 2]
            rr = ring_send(src_r, comm_r.at[s % 2],
                           send_r_sem, recv_r_sem, right)
            rl = ring_send(src_l, comm_l.at[s % 2],
                           send_l_sem, recv_l_sem, left)
            rr.start()
            rl.start()
            if g == 0:
                wsrc_r, wslab_r = slab_r.at[1], lax.rem(me + 1, N_DEV)
                wsrc_l, wslab_l = slab_l.at[1], lax.rem(me + N_DEV - 1, N_DEV)
            else:
                wsrc_r = comm_r.at[(s - 1) % 2]
                wslab_r = lax.rem(me + N_DEV - (g - 1), N_DEV)
                wsrc_l = comm_l.at[(s - 1) % 2]
                wslab_l = lax.rem(me + g - 1, N_DEV)
            cr = pltpu.make_async_copy(
                wsrc_r, out_hbm.at[0, pl.ds(wslab_r * SLAB, SLAB), :],
                out_r_sem)
            cl = pltpu.make_async_copy(
                wsrc_l, out_hbm.at[1, pl.ds(wslab_l * SLAB, SLAB), :],
                out_l_sem)
            cr.start()
            cl.start()
            rr.wait()
            rl.wait()
            cr.wait()
            cl.wait()
            if s == 4:
                send_credits()
        cr = pltpu.make_async_copy(
            comm_r.at[1],
            out_hbm.at[0, pl.ds(lax.rem(me + 2, N_DEV) * SLAB, SLAB), :],
            out_r_sem)
        cl = pltpu.make_async_copy(
            comm_l.at[1],
            out_hbm.at[1, pl.ds(lax.rem(me + 2, N_DEV) * SLAB, SLAB), :],
            out_l_sem)
        cr.start()
        cl.start()
        cr.wait()
        cl.wait()

    return pl.pallas_call(
        body,
        out_shape=jax.ShapeDtypeStruct((B, S, O), jnp.float32),
        in_specs=[
            pl.BlockSpec(memory_space=pltpu.MemorySpace.HBM),
            pl.BlockSpec(memory_space=pltpu.MemorySpace.VMEM),
            pl.BlockSpec(memory_space=pltpu.MemorySpace.VMEM),
        ],
        out_specs=pl.BlockSpec(memory_space=pltpu.MemorySpace.HBM),
        scratch_shapes=[
            pltpu.VMEM((SLAB + HALO, C), jnp.float32),
            pltpu.VMEM((2, SLAB, O), jnp.float32),
            pltpu.VMEM((2, SLAB, O), jnp.float32),
            pltpu.VMEM((2, SLAB, O), jnp.float32),
            pltpu.VMEM((2, SLAB, O), jnp.float32),
            pltpu.SemaphoreType.DMA,
            pltpu.SemaphoreType.DMA,
            pltpu.SemaphoreType.DMA,
            pltpu.SemaphoreType.DMA,
            pltpu.SemaphoreType.DMA,
            pltpu.SemaphoreType.DMA,
            pltpu.SemaphoreType.DMA,
            pltpu.SemaphoreType.REGULAR,
            pltpu.SemaphoreType.REGULAR,
        ],
        compiler_params=pltpu.CompilerParams(
            collective_id=0,
            vmem_limit_bytes=58 * 1024 * 1024,
        ),
    )(x, k, Wp)
